# Initial kernel scaffold; baseline (speedup 1.0000x reference)
import jax
import jax.numpy as jnp
from jax.experimental import pallas as pl


def kernel(x, batch, W1, b1, W2, b2):
    raise NotImplementedError("write your pallas kernel here")


# speedup vs baseline: 17.0239x; 17.0239x over previous
import functools

import jax
import jax.numpy as jnp
from jax.experimental import pallas as pl
from jax.experimental.pallas import tpu as pltpu


_BLK = 4000
_WIN = 64
_OUT_S = 1024


def _pool_kernel(blkinfo_ref, b2_ref, batch_ref, x_ref, w1_ref, b1_ref,
                 w2_ref, out_ref, acc_ref, m_ref, l_ref,
                 *, nsteps, s_out, win):
    k = pl.program_id(0)

    @pl.when(k == 0)
    def _init():
        acc_ref[:, :] = jnp.zeros_like(acc_ref)
        m_ref[:, :] = jnp.zeros_like(m_ref)
        l_ref[:, :] = jnp.zeros_like(l_ref)

    x_blk = x_ref[:, :]
    ids = batch_ref[0]

    h = jax.lax.dot_general(x_blk, w1_ref[:, :], (((1,), (1,)), ((), ())),
                            preferred_element_type=jnp.float32)
    h = jnp.maximum(h + b1_ref[:, :], 0.0)
    gate = jax.lax.dot_general(w2_ref[:, :], h, (((1,), (1,)), ((), ())),
                               preferred_element_type=jnp.float32)
    gate = gate + b2_ref[0]

    start0 = blkinfo_ref[k, 0]
    nwin = (blkinfo_ref[k, 1] - start0) // win + 1

    x_bf = x_blk.astype(jnp.bfloat16)

    def body(j, carry):
        start = start0 + j * win
        rows = start + jax.lax.broadcasted_iota(jnp.int32, (win, 1), 0)
        hit = ids == rows
        m01 = hit.astype(jnp.float32)
        bmax = jnp.max(jnp.where(hit, gate, -1e30), axis=1, keepdims=True)
        m_old = m_ref[pl.ds(start, win), :]
        m_new = jnp.maximum(m_old, bmax)
        alpha = jnp.exp(m_old - m_new)
        m_ref[pl.ds(start, win), :] = m_new
        m_row = jnp.sum(m01 * m_new, axis=0, keepdims=True)
        in_w = (ids >= start) & (ids < start + win)
        e = jnp.where(in_w, jnp.exp(gate - m_row), 0.0)
        p = m01 * e
        l_add = jnp.sum(p, axis=1, keepdims=True)
        l_ref[pl.ds(start, win), :] = (
            l_ref[pl.ds(start, win), :] * alpha + l_add)
        contrib = jax.lax.dot_general(
            p.astype(jnp.bfloat16), x_bf, (((1,), (0,)), ((), ())),
            preferred_element_type=jnp.float32)
        acc_ref[pl.ds(start, win), :] = (
            acc_ref[pl.ds(start, win), :] * alpha + contrib)
        return carry

    jax.lax.fori_loop(0, nwin, body, 0)

    @pl.when(k == nsteps - 1)
    def _fin():
        out_ref[:, :] = acc_ref[0:s_out, :] / (l_ref[0:s_out, :] + 1e-9)


def _run(x, batch, W1, b1, W2, b2, s_out, blk, win, interpret=False):
    n, d = x.shape
    hdim = W1.shape[0]
    nsteps = n // blk
    assert nsteps * blk == n
    s_pad = s_out + 2 * win

    batch3 = batch.reshape(nsteps, 1, blk)
    first = (batch3[:, 0, 0] // 8) * 8
    last = batch3[:, 0, blk - 1]
    blkinfo = jnp.stack([first, last], axis=1).astype(jnp.int32)

    in_specs = [
            pl.BlockSpec(memory_space=pltpu.SMEM),
            pl.BlockSpec(memory_space=pltpu.SMEM),
            pl.BlockSpec((1, 1, blk), lambda k: (k, 0, 0)),
            pl.BlockSpec((blk, d), lambda k: (k, 0)),
            pl.BlockSpec((hdim, d), lambda k: (0, 0)),
            pl.BlockSpec((1, hdim), lambda k: (0, 0)),
            pl.BlockSpec((1, hdim), lambda k: (0, 0)),
    ]

    fn = pl.pallas_call(
        functools.partial(_pool_kernel, nsteps=nsteps, s_out=s_out, win=win),
        grid=(nsteps,),
        in_specs=in_specs,
        out_specs=pl.BlockSpec((s_out, d), lambda k: (0, 0)),
        out_shape=jax.ShapeDtypeStruct((s_out, d), x.dtype),
        scratch_shapes=[
            pltpu.VMEM((s_pad, d), jnp.float32),
            pltpu.VMEM((s_pad, 1), jnp.float32),
            pltpu.VMEM((s_pad, 1), jnp.float32),
        ],
        compiler_params=pltpu.CompilerParams(
            dimension_semantics=("arbitrary",)),
        interpret=interpret,
    )
    return fn(blkinfo, b2, batch3, x, W1, b1.reshape(1, hdim), W2)


def kernel(x, batch, W1, b1, W2, b2):
    return _run(x, batch, W1, b1, W2, b2, _OUT_S, _BLK, _WIN)


# speedup vs baseline: 17.0614x; 1.0022x over previous
import functools

import jax
import jax.numpy as jnp
from jax.experimental import pallas as pl
from jax.experimental.pallas import tpu as pltpu


_BLK = 4000
_WIN = 64
_OUT_S = 1024


def _pool_kernel(blkinfo_ref, b2_ref, batch_ref, x_ref, w1_ref, b1_ref,
                 w2_ref, out_ref, acc_ref, m_ref, l_ref,
                 *, nsteps, s_out, win):
    k = pl.program_id(0)

    @pl.when(k == 0)
    def _init():
        acc_ref[:, :] = jnp.zeros_like(acc_ref)
        m_ref[:, :] = jnp.zeros_like(m_ref)
        l_ref[:, :] = jnp.zeros_like(l_ref)

    x_blk = x_ref[:, :]
    ids = batch_ref[0]
    x_bf = x_blk.astype(jnp.bfloat16)

    h = jax.lax.dot_general(x_bf, w1_ref[:, :].astype(jnp.bfloat16),
                            (((1,), (1,)), ((), ())),
                            preferred_element_type=jnp.float32)
    h = jnp.maximum(h + b1_ref[:, :], 0.0)
    gate = jax.lax.dot_general(w2_ref[:, :].astype(jnp.bfloat16),
                               h.astype(jnp.bfloat16),
                               (((1,), (1,)), ((), ())),
                               preferred_element_type=jnp.float32)
    gate = gate + b2_ref[0]

    start0 = blkinfo_ref[k, 0]
    nwin = (blkinfo_ref[k, 1] - start0) // win + 1

    def body(j, carry):
        start = start0 + j * win
        rows = start + jax.lax.broadcasted_iota(jnp.int32, (win, 1), 0)
        hit = ids == rows
        m01 = hit.astype(jnp.float32)
        bmax = jnp.max(jnp.where(hit, gate, -1e30), axis=1, keepdims=True)
        m_old = m_ref[pl.ds(start, win), :]
        m_new = jnp.maximum(m_old, bmax)
        alpha = jnp.exp(m_old - m_new)
        m_ref[pl.ds(start, win), :] = m_new
        m_row = jnp.sum(m01 * m_new, axis=0, keepdims=True)
        in_w = (ids >= start) & (ids < start + win)
        e = jnp.where(in_w, jnp.exp(gate - m_row), 0.0)
        p = m01 * e
        l_add = jnp.sum(p, axis=1, keepdims=True)
        l_ref[pl.ds(start, win), :] = (
            l_ref[pl.ds(start, win), :] * alpha + l_add)
        contrib = jax.lax.dot_general(
            p.astype(jnp.bfloat16), x_bf, (((1,), (0,)), ((), ())),
            preferred_element_type=jnp.float32)
        acc_ref[pl.ds(start, win), :] = (
            acc_ref[pl.ds(start, win), :] * alpha + contrib)
        return carry

    jax.lax.fori_loop(0, nwin, body, 0)

    @pl.when(k == nsteps - 1)
    def _fin():
        out_ref[:, :] = acc_ref[0:s_out, :] / (l_ref[0:s_out, :] + 1e-9)


def _run(x, batch, W1, b1, W2, b2, s_out, blk, win, interpret=False):
    n, d = x.shape
    hdim = W1.shape[0]
    nsteps = n // blk
    assert nsteps * blk == n
    s_pad = s_out + 2 * win

    batch3 = batch.reshape(nsteps, 1, blk)
    first = (batch3[:, 0, 0] // 8) * 8
    last = batch3[:, 0, blk - 1]
    blkinfo = jnp.stack([first, last], axis=1).astype(jnp.int32)

    in_specs = [
            pl.BlockSpec(memory_space=pltpu.SMEM),
            pl.BlockSpec(memory_space=pltpu.SMEM),
            pl.BlockSpec((1, 1, blk), lambda k: (k, 0, 0)),
            pl.BlockSpec((blk, d), lambda k: (k, 0)),
            pl.BlockSpec((hdim, d), lambda k: (0, 0)),
            pl.BlockSpec((1, hdim), lambda k: (0, 0)),
            pl.BlockSpec((1, hdim), lambda k: (0, 0)),
    ]

    fn = pl.pallas_call(
        functools.partial(_pool_kernel, nsteps=nsteps, s_out=s_out, win=win),
        grid=(nsteps,),
        in_specs=in_specs,
        out_specs=pl.BlockSpec((s_out, d), lambda k: (0, 0)),
        out_shape=jax.ShapeDtypeStruct((s_out, d), x.dtype),
        scratch_shapes=[
            pltpu.VMEM((s_pad, d), jnp.float32),
            pltpu.VMEM((s_pad, 1), jnp.float32),
            pltpu.VMEM((s_pad, 1), jnp.float32),
        ],
        compiler_params=pltpu.CompilerParams(
            dimension_semantics=("arbitrary",)),
        interpret=interpret,
    )
    return fn(blkinfo, b2, batch3, x, W1, b1.reshape(1, hdim), W2)


def kernel(x, batch, W1, b1, W2, b2):
    return _run(x, batch, W1, b1, W2, b2, _OUT_S, _BLK, _WIN)


# speedup vs baseline: 17.1379x; 1.0045x over previous
import functools

import jax
import jax.numpy as jnp
from jax.experimental import pallas as pl
from jax.experimental.pallas import tpu as pltpu


_BLK = 4000
_WIN = 64
_OUT_S = 1024


def _pool_kernel(blkinfo_ref, b2_ref, batch_ref, x_ref, w1_ref, b1_ref,
                 w2_ref, out_ref, acc_ref, m_ref, l_ref,
                 *, nsteps, s_out, win):
    k = pl.program_id(0)

    @pl.when(k == 0)
    def _init():
        acc_ref[:, :] = jnp.zeros_like(acc_ref)
        m_ref[:, :] = jnp.zeros_like(m_ref)
        l_ref[:, :] = jnp.zeros_like(l_ref)

    x_blk = x_ref[:, :]
    ids = batch_ref[0]
    x_bf = x_blk.astype(jnp.bfloat16)

    h = jax.lax.dot_general(x_bf, w1_ref[:, :].astype(jnp.bfloat16),
                            (((1,), (1,)), ((), ())),
                            preferred_element_type=jnp.float32)
    h = jnp.maximum(h + b1_ref[:, :], 0.0)
    gate = jax.lax.dot_general(w2_ref[:, :].astype(jnp.bfloat16),
                               h.astype(jnp.bfloat16),
                               (((1,), (1,)), ((), ())),
                               preferred_element_type=jnp.float32)
    gate = gate + b2_ref[0]

    start0 = blkinfo_ref[k, 0]
    nwin = (blkinfo_ref[k, 1] - start0) // win + 1

    def body(j, carry):
        start = start0 + j * win
        rows = start + jax.lax.broadcasted_iota(jnp.int32, (win, 1), 0)
        hit = ids == rows
        bmax = jnp.max(jnp.where(hit, gate, -1e30), axis=1, keepdims=True)
        m_old = m_ref[pl.ds(start, win), :]
        m_new = jnp.maximum(m_old, bmax)
        alpha = jnp.exp(m_old - m_new)
        m_ref[pl.ds(start, win), :] = m_new
        m_row = jnp.sum(jnp.where(hit, m_new, 0.0), axis=0, keepdims=True)
        in_w = (ids >= start) & (ids < start + win)
        e = jnp.where(in_w, jnp.exp(gate - m_row), 0.0)
        p = jnp.where(hit, e, 0.0)
        l_add = jnp.sum(p, axis=1, keepdims=True)
        l_ref[pl.ds(start, win), :] = (
            l_ref[pl.ds(start, win), :] * alpha + l_add)
        contrib = jax.lax.dot_general(
            p.astype(jnp.bfloat16), x_bf, (((1,), (0,)), ((), ())),
            preferred_element_type=jnp.float32)
        acc_ref[pl.ds(start, win), :] = (
            acc_ref[pl.ds(start, win), :] * alpha + contrib)
        return carry

    jax.lax.fori_loop(0, nwin, body, 0)

    @pl.when(k == nsteps - 1)
    def _fin():
        out_ref[:, :] = acc_ref[0:s_out, :] / (l_ref[0:s_out, :] + 1e-9)


def _run(x, batch, W1, b1, W2, b2, s_out, blk, win, interpret=False):
    n, d = x.shape
    hdim = W1.shape[0]
    nsteps = n // blk
    assert nsteps * blk == n
    s_pad = s_out + 2 * win

    batch3 = batch.reshape(nsteps, 1, blk)
    first = (batch3[:, 0, 0] // 8) * 8
    last = batch3[:, 0, blk - 1]
    blkinfo = jnp.stack([first, last], axis=1).astype(jnp.int32)

    in_specs = [
            pl.BlockSpec(memory_space=pltpu.SMEM),
            pl.BlockSpec(memory_space=pltpu.SMEM),
            pl.BlockSpec((1, 1, blk), lambda k: (k, 0, 0)),
            pl.BlockSpec((blk, d), lambda k: (k, 0)),
            pl.BlockSpec((hdim, d), lambda k: (0, 0)),
            pl.BlockSpec((1, hdim), lambda k: (0, 0)),
            pl.BlockSpec((1, hdim), lambda k: (0, 0)),
    ]

    fn = pl.pallas_call(
        functools.partial(_pool_kernel, nsteps=nsteps, s_out=s_out, win=win),
        grid=(nsteps,),
        in_specs=in_specs,
        out_specs=pl.BlockSpec((s_out, d), lambda k: (0, 0)),
        out_shape=jax.ShapeDtypeStruct((s_out, d), x.dtype),
        scratch_shapes=[
            pltpu.VMEM((s_pad, d), jnp.float32),
            pltpu.VMEM((s_pad, 1), jnp.float32),
            pltpu.VMEM((s_pad, 1), jnp.float32),
        ],
        compiler_params=pltpu.CompilerParams(
            dimension_semantics=("arbitrary",)),
        interpret=interpret,
    )
    return fn(blkinfo, b2, batch3, x, W1, b1.reshape(1, hdim), W2)


def kernel(x, batch, W1, b1, W2, b2):
    return _run(x, batch, W1, b1, W2, b2, _OUT_S, _BLK, _WIN)


# speedup vs baseline: 21.4470x; 1.2514x over previous
import functools

import jax
import jax.numpy as jnp
from jax.experimental import pallas as pl
from jax.experimental.pallas import tpu as pltpu


_BLK = 8000
_WIN = 64
_OUT_S = 1024


def _pool_kernel(blkinfo_ref, b2_ref, batch_ref, x_ref, w1_ref, b1_ref,
                 w2_ref, out_ref, acc_ref, m_ref, l_ref,
                 *, nsteps, s_out, win):
    k = pl.program_id(0)

    @pl.when(k == 0)
    def _init():
        acc_ref[:, :] = jnp.zeros_like(acc_ref)
        m_ref[:, :] = jnp.zeros_like(m_ref)
        l_ref[:, :] = jnp.zeros_like(l_ref)

    x_blk = x_ref[:, :]
    ids = batch_ref[0]
    x_bf = x_blk.astype(jnp.bfloat16)

    h = jax.lax.dot_general(x_bf, w1_ref[:, :].astype(jnp.bfloat16),
                            (((1,), (1,)), ((), ())),
                            preferred_element_type=jnp.float32)
    h = jnp.maximum(h + b1_ref[:, :], 0.0)
    gate = jax.lax.dot_general(w2_ref[:, :].astype(jnp.bfloat16),
                               h.astype(jnp.bfloat16),
                               (((1,), (1,)), ((), ())),
                               preferred_element_type=jnp.float32)
    gate = gate + b2_ref[0]

    start0 = blkinfo_ref[k, 0]
    nwin = (blkinfo_ref[k, 1] - start0) // win + 1

    def body(j, carry):
        start = start0 + j * win
        rows = start + jax.lax.broadcasted_iota(jnp.int32, (win, 1), 0)
        hit = ids == rows
        bmax = jnp.max(jnp.where(hit, gate, -1e30), axis=1, keepdims=True)
        m_old = m_ref[pl.ds(start, win), :]
        m_new = jnp.maximum(m_old, bmax)
        alpha = jnp.exp(m_old - m_new)
        m_ref[pl.ds(start, win), :] = m_new
        m_row = jnp.sum(jnp.where(hit, m_new, 0.0), axis=0, keepdims=True)
        in_w = (ids >= start) & (ids < start + win)
        e = jnp.where(in_w, jnp.exp(gate - m_row), 0.0)
        p = jnp.where(hit, e, 0.0)
        l_add = jnp.sum(p, axis=1, keepdims=True)
        l_ref[pl.ds(start, win), :] = (
            l_ref[pl.ds(start, win), :] * alpha + l_add)
        contrib = jax.lax.dot_general(
            p.astype(jnp.bfloat16), x_bf, (((1,), (0,)), ((), ())),
            preferred_element_type=jnp.float32)
        acc_ref[pl.ds(start, win), :] = (
            acc_ref[pl.ds(start, win), :] * alpha + contrib)
        return carry

    jax.lax.fori_loop(0, nwin, body, 0)

    @pl.when(k == nsteps - 1)
    def _fin():
        out_ref[:, :] = acc_ref[0:s_out, :] / (l_ref[0:s_out, :] + 1e-9)


def _run(x, batch, W1, b1, W2, b2, s_out, blk, win, interpret=False):
    n, d = x.shape
    hdim = W1.shape[0]
    nsteps = n // blk
    assert nsteps * blk == n
    s_pad = s_out + 2 * win

    batch3 = batch.reshape(nsteps, 1, blk)
    first = (batch3[:, 0, 0] // 8) * 8
    last = batch3[:, 0, blk - 1]
    blkinfo = jnp.stack([first, last], axis=1).astype(jnp.int32)

    in_specs = [
            pl.BlockSpec(memory_space=pltpu.SMEM),
            pl.BlockSpec(memory_space=pltpu.SMEM),
            pl.BlockSpec((1, 1, blk), lambda k: (k, 0, 0)),
            pl.BlockSpec((blk, d), lambda k: (k, 0)),
            pl.BlockSpec((hdim, d), lambda k: (0, 0)),
            pl.BlockSpec((1, hdim), lambda k: (0, 0)),
            pl.BlockSpec((1, hdim), lambda k: (0, 0)),
    ]

    fn = pl.pallas_call(
        functools.partial(_pool_kernel, nsteps=nsteps, s_out=s_out, win=win),
        grid=(nsteps,),
        in_specs=in_specs,
        out_specs=pl.BlockSpec((s_out, d), lambda k: (0, 0)),
        out_shape=jax.ShapeDtypeStruct((s_out, d), x.dtype),
        scratch_shapes=[
            pltpu.VMEM((s_pad, d), jnp.float32),
            pltpu.VMEM((s_pad, 1), jnp.float32),
            pltpu.VMEM((s_pad, 1), jnp.float32),
        ],
        compiler_params=pltpu.CompilerParams(
            dimension_semantics=("arbitrary",)),
        interpret=interpret,
    )
    return fn(blkinfo, b2, batch3, x, W1, b1.reshape(1, hdim), W2)


def kernel(x, batch, W1, b1, W2, b2):
    return _run(x, batch, W1, b1, W2, b2, _OUT_S, _BLK, _WIN)


# speedup vs baseline: 23.6194x; 1.1013x over previous
import functools

import jax
import jax.numpy as jnp
from jax.experimental import pallas as pl
from jax.experimental.pallas import tpu as pltpu


_BLK = 16000
_WIN = 64
_OUT_S = 1024


def _pool_kernel(blkinfo_ref, b2_ref, batch_ref, x_ref, w1_ref, b1_ref,
                 w2_ref, out_ref, acc_ref, m_ref, l_ref,
                 *, nsteps, s_out, win):
    k = pl.program_id(0)

    @pl.when(k == 0)
    def _init():
        acc_ref[:, :] = jnp.zeros_like(acc_ref)
        m_ref[:, :] = jnp.zeros_like(m_ref)
        l_ref[:, :] = jnp.zeros_like(l_ref)

    x_blk = x_ref[:, :]
    ids = batch_ref[0]
    x_bf = x_blk.astype(jnp.bfloat16)

    h = jax.lax.dot_general(x_bf, w1_ref[:, :].astype(jnp.bfloat16),
                            (((1,), (1,)), ((), ())),
                            preferred_element_type=jnp.float32)
    h = jnp.maximum(h + b1_ref[:, :], 0.0)
    gate = jax.lax.dot_general(w2_ref[:, :].astype(jnp.bfloat16),
                               h.astype(jnp.bfloat16),
                               (((1,), (1,)), ((), ())),
                               preferred_element_type=jnp.float32)
    gate = gate + b2_ref[0]

    start0 = blkinfo_ref[k, 0]
    nwin = (blkinfo_ref[k, 1] - start0) // win + 1

    def body(j, carry):
        start = start0 + j * win
        rows = start + jax.lax.broadcasted_iota(jnp.int32, (win, 1), 0)
        hit = ids == rows
        bmax = jnp.max(jnp.where(hit, gate, -1e30), axis=1, keepdims=True)
        m_old = m_ref[pl.ds(start, win), :]
        m_new = jnp.maximum(m_old, bmax)
        alpha = jnp.exp(m_old - m_new)
        m_ref[pl.ds(start, win), :] = m_new
        m_row = jnp.sum(jnp.where(hit, m_new, 0.0), axis=0, keepdims=True)
        in_w = (ids >= start) & (ids < start + win)
        e = jnp.where(in_w, jnp.exp(gate - m_row), 0.0)
        p = jnp.where(hit, e, 0.0)
        l_add = jnp.sum(p, axis=1, keepdims=True)
        l_ref[pl.ds(start, win), :] = (
            l_ref[pl.ds(start, win), :] * alpha + l_add)
        contrib = jax.lax.dot_general(
            p.astype(jnp.bfloat16), x_bf, (((1,), (0,)), ((), ())),
            preferred_element_type=jnp.float32)
        acc_ref[pl.ds(start, win), :] = (
            acc_ref[pl.ds(start, win), :] * alpha + contrib)
        return carry

    jax.lax.fori_loop(0, nwin, body, 0)

    @pl.when(k == nsteps - 1)
    def _fin():
        out_ref[:, :] = acc_ref[0:s_out, :] / (l_ref[0:s_out, :] + 1e-9)


def _run(x, batch, W1, b1, W2, b2, s_out, blk, win, interpret=False):
    n, d = x.shape
    hdim = W1.shape[0]
    nsteps = n // blk
    assert nsteps * blk == n
    s_pad = s_out + 2 * win

    batch3 = batch.reshape(nsteps, 1, blk)
    first = (batch3[:, 0, 0] // 8) * 8
    last = batch3[:, 0, blk - 1]
    blkinfo = jnp.stack([first, last], axis=1).astype(jnp.int32)

    in_specs = [
            pl.BlockSpec(memory_space=pltpu.SMEM),
            pl.BlockSpec(memory_space=pltpu.SMEM),
            pl.BlockSpec((1, 1, blk), lambda k: (k, 0, 0)),
            pl.BlockSpec((blk, d), lambda k: (k, 0)),
            pl.BlockSpec((hdim, d), lambda k: (0, 0)),
            pl.BlockSpec((1, hdim), lambda k: (0, 0)),
            pl.BlockSpec((1, hdim), lambda k: (0, 0)),
    ]

    fn = pl.pallas_call(
        functools.partial(_pool_kernel, nsteps=nsteps, s_out=s_out, win=win),
        grid=(nsteps,),
        in_specs=in_specs,
        out_specs=pl.BlockSpec((s_out, d), lambda k: (0, 0)),
        out_shape=jax.ShapeDtypeStruct((s_out, d), x.dtype),
        scratch_shapes=[
            pltpu.VMEM((s_pad, d), jnp.float32),
            pltpu.VMEM((s_pad, 1), jnp.float32),
            pltpu.VMEM((s_pad, 1), jnp.float32),
        ],
        compiler_params=pltpu.CompilerParams(
            dimension_semantics=("arbitrary",)),
        interpret=interpret,
    )
    return fn(blkinfo, b2, batch3, x, W1, b1.reshape(1, hdim), W2)


def kernel(x, batch, W1, b1, W2, b2):
    return _run(x, batch, W1, b1, W2, b2, _OUT_S, _BLK, _WIN)
